# baseline (speedup 1.0000x reference)
import jax
import jax.numpy as jnp
from jax import lax
from jax.experimental import pallas as pl
from jax.experimental.pallas import tpu as pltpu
from jax.experimental.pallas import tpu_sc as plsc

_LANES = 16


def _row_check_body(rs_hbm, out_hbm, rs_v, buf_v, out_v):
    c = lax.axis_index("c")
    s = lax.axis_index("s")

    @pl.when(jnp.logical_and(c == 0, s == 0))
    def _():
        pltpu.sync_copy(rs_hbm, rs_v)
        lo = rs_v[pl.ds(0, _LANES)]
        hi = rs_v[pl.ds(1, _LANES)]
        rt_row_lengths = hi - lo
        rs_row_lengths = hi - lo
        eq = jnp.where(
            rt_row_lengths == rs_row_lengths,
            jnp.ones((_LANES,), jnp.int32),
            jnp.zeros((_LANES,), jnp.int32),
        )
        buf_v[pl.ds(_LANES, _LANES)] = jnp.ones((_LANES,), jnp.int32)
        buf_v[pl.ds(0, _LANES)] = eq
        for off in (8, 4, 2, 1):
            buf_v[pl.ds(0, _LANES)] = (
                buf_v[pl.ds(0, _LANES)] & buf_v[pl.ds(off, _LANES)]
            )
        out_v[...] = buf_v[pl.ds(0, _LANES)]
        pltpu.sync_copy(out_v, out_hbm)


def _row_lengths_equal_sc(row_splits):
    mesh = plsc.VectorSubcoreMesh(core_axis_name="c", subcore_axis_name="s")
    flags = pl.kernel(
        _row_check_body,
        out_type=jax.ShapeDtypeStruct((_LANES,), jnp.int32),
        mesh=mesh,
        scratch_types=[
            pltpu.VMEM((_LANES + 1,), jnp.int32),
            pltpu.VMEM((2 * _LANES,), jnp.int32),
            pltpu.VMEM((_LANES,), jnp.int32),
        ],
    )(row_splits)
    return flags[0].astype(jnp.bool_)


def kernel(values, row_splits):
    return (values, row_splits, _row_lengths_equal_sc(row_splits))
